# baseline (speedup 1.0000x reference)
import functools

import jax
import jax.numpy as jnp
import numpy as np
from jax import lax
from jax.experimental import pallas as pl
from jax.experimental.pallas import tpu as pltpu
from jax.experimental.pallas import tpu_sc as plsc

FRAME_RANGE = 5
NUM_POS_FEATS = 32
DIM_METADATA = 1
TEMPERATURE = 10000.0
B = 16384
EMBED = FRAME_RANGE * 4 * NUM_POS_FEATS
NMETA = FRAME_RANGE * DIM_METADATA
OUT_W = EMBED + NMETA
ROWS_PER_BLOCK = 512

_NC, _NS, _L = 2, 16, 16
_NW = _NC * _NS
_M = 1000000
_RNG = 31264
_MPAD = _RNG * _NW
_NQ = B // _NW
_QCH = 128

_mesh = plsc.VectorSubcoreMesh(
    core_axis_name="c", subcore_axis_name="s", num_cores=_NC, num_subcores=_NS)


def _wid():
    return lax.axis_index("s") * _NC + lax.axis_index("c")


def _posmap_body(w_hbm, pmap_hbm, wv, buf, sem):
    wid = _wid()
    lo = wid * _RNG
    pltpu.async_copy(w_hbm, wv, sem).wait()
    lane = lax.iota(jnp.int32, _L)
    zeros = jnp.zeros((_L,), jnp.int32)
    onehot = jnp.int32(1) << lane

    def memset_it(r, c):
        buf[pl.ds(pl.multiple_of(r * _L, _L), _L)] = zeros
        return c

    lax.fori_loop(0, _RNG // _L, memset_it, 0)

    def scat_it(jc, c):
        wvv = wv[pl.ds(pl.multiple_of(jc * _L, _L), _L)]
        rel = wvv - lo
        valid = (rel >= 0) & (rel < _RNG)
        rel = jnp.where(valid, rel, 0)
        pk = (zeros + jc + 1) * 65536 + onehot
        plsc.store_scatter(buf, [rel], pk, mask=valid)
        return c

    lax.fori_loop(0, B // _L, scat_it, 0)
    pltpu.sync_copy(buf, pmap_hbm.at[pl.ds(lo, _RNG)])


@functools.partial(
    pl.kernel,
    out_type=jax.ShapeDtypeStruct((_MPAD,), jnp.int32),
    mesh=_mesh,
    compiler_params=pltpu.CompilerParams(needs_layout_passes=False, use_tc_tiling_on_sc=False),
    scratch_types=[
        pltpu.VMEM((B,), jnp.int32),
        pltpu.VMEM((_RNG,), jnp.int32),
        pltpu.SemaphoreType.DMA,
    ],
)
def _posmap(w_hbm, pmap_hbm, wv, buf, sem):
    _posmap_body(w_hbm, pmap_hbm, wv, buf, sem)


def _gatherq_body(pmap_hbm, q_hbm, trk_hbm, meta_hbm, det_hbm,
                  posq_hbm, trkg_hbm, metag_hbm, detg_hbm,
                  qv, cmv, posv, pcv, trkidx, metaidx, trkvals, metavals, detv,
                  sem0, sem1, sem2, sem3):
    wid = _wid()
    base = wid * _NQ
    nch = _NQ // _QCH
    pltpu.async_copy(q_hbm.at[pl.ds(base, _NQ)], qv, sem0).wait()
    pos_cps = [
        pltpu.async_copy(
            pmap_hbm.at[qv.at[pl.ds(c * _QCH, _QCH)]],
            cmv.at[pl.ds(c * _QCH, _QCH)], sem0)
        for c in range(nch)
    ]

    lane = lax.iota(jnp.int32, _L)

    def idx_it(c, carry):
        qs = qv[pl.ds(pl.multiple_of(c * _L, _L), _L)]
        b20 = (lane + c * _L) * 20
        q20 = qs * 20
        for k in range(20):
            plsc.store_scatter(trkidx, [b20 + k], q20 + k)
        b5 = (lane + c * _L) * 5
        q5 = qs * 5
        for k in range(5):
            plsc.store_scatter(metaidx, [b5 + k], q5 + k)
        return carry

    lax.fori_loop(0, _NQ // _L, idx_it, 0)
    trk_cps = [
        pltpu.async_copy(
            trk_hbm.at[trkidx.at[pl.ds(c * _QCH, _QCH)]],
            trkvals.at[pl.ds(c * _QCH, _QCH)], sem1)
        for c in range(20 * _NQ // _QCH)
    ]
    meta_cps = [
        pltpu.async_copy(
            meta_hbm.at[metaidx.at[pl.ds(c * _QCH, _QCH)]],
            metavals.at[pl.ds(c * _QCH, _QCH)], sem2)
        for c in range(5 * _NQ // _QCH)
    ]
    for cp in pos_cps:
        cp.wait()

    def decode_it(c, carry):
        sl = pl.ds(pl.multiple_of(c * _L, _L), _L)
        pk = cmv[sl]
        jc = (pk >> 16) - 1
        m = pk & 0xFFFF
        mf = m.astype(jnp.float32)
        hb = (lax.bitcast_convert_type(mf, jnp.int32) >> 23) - 127
        pos = jnp.where(jc >= 0, jc * _L + hb, -1)
        posv[sl] = pos
        pcv[sl] = jnp.maximum(pos, 0)
        return carry

    lax.fori_loop(0, _NQ // _L, decode_it, 0)
    det_cps = [
        pltpu.async_copy(
            det_hbm.at[pcv.at[pl.ds(c * _QCH, _QCH)]],
            detv.at[pl.ds(c * _QCH, _QCH)], sem3)
        for c in range(nch)
    ]
    pltpu.sync_copy(posv, posq_hbm.at[pl.ds(base, _NQ)])
    for cp in trk_cps:
        cp.wait()
    pltpu.sync_copy(trkvals, trkg_hbm.at[pl.ds(base * 20, _NQ * 20)])
    for cp in meta_cps:
        cp.wait()
    pltpu.sync_copy(metavals, metag_hbm.at[pl.ds(base * 5, _NQ * 5)])
    for cp in det_cps:
        cp.wait()
    pltpu.sync_copy(detv, detg_hbm.at[pl.ds(base, _NQ)])


@functools.partial(
    pl.kernel,
    out_type=(
        jax.ShapeDtypeStruct((B,), jnp.int32),
        jax.ShapeDtypeStruct((B * 20,), jnp.float32),
        jax.ShapeDtypeStruct((B * 5,), jnp.float32),
        jax.ShapeDtypeStruct((B, 8), jnp.float32),
    ),
    mesh=_mesh,
    compiler_params=pltpu.CompilerParams(needs_layout_passes=False, use_tc_tiling_on_sc=False),
    scratch_types=[
        pltpu.VMEM((_NQ,), jnp.int32),
        pltpu.VMEM((_NQ,), jnp.int32),
        pltpu.VMEM((_NQ,), jnp.int32),
        pltpu.VMEM((_NQ,), jnp.int32),
        pltpu.VMEM((_NQ * 20,), jnp.int32),
        pltpu.VMEM((_NQ * 5,), jnp.int32),
        pltpu.VMEM((_NQ * 20,), jnp.float32),
        pltpu.VMEM((_NQ * 5,), jnp.float32),
        pltpu.VMEM((_NQ, 8), jnp.float32),
        pltpu.SemaphoreType.DMA,
        pltpu.SemaphoreType.DMA,
        pltpu.SemaphoreType.DMA,
        pltpu.SemaphoreType.DMA,
    ],
)
def _gatherq(pmap_hbm, q_hbm, trk_hbm, meta_hbm, det_hbm,
             posq_hbm, trkg_hbm, metag_hbm, detg_hbm,
             qv, cmv, posv, pcv, trkidx, metaidx, trkvals, metavals, detv,
             sem0, sem1, sem2, sem3):
    _gatherq_body(pmap_hbm, q_hbm, trk_hbm, meta_hbm, det_hbm,
                  posq_hbm, trkg_hbm, metag_hbm, detg_hbm,
                  qv, cmv, posv, pcv, trkidx, metaidx, trkvals, metavals, detv,
                  sem0, sem1, sem2, sem3)


def _freq_matrices():
    i = np.arange(NUM_POS_FEATS // 2, dtype=np.float64)
    w = 2.0 * np.pi * TEMPERATURE ** (-i / (NUM_POS_FEATS // 2))
    w2 = np.concatenate([w, w])
    ngroups = FRAME_RANGE * 4
    S = np.zeros((ngroups, EMBED), dtype=np.float32)
    for g in range(ngroups):
        S[g, g * 32:(g + 1) * 32] = w2
    S4 = S.reshape(FRAME_RANGE, 4, EMBED).sum(axis=0).astype(np.float32)
    shift = np.tile(
        np.concatenate([np.zeros(16), np.full(16, np.pi / 2.0)]), ngroups
    ).astype(np.float32)
    return S, S4, shift[None, :]


_S, _S4, _SHIFT = _freq_matrices()


def _encode_body(trkg_ref, metag_ref, detg_ref, posq_ref,
                 s_ref, s4_ref, shift_ref, out_ref):
    sel = posq_ref[...] >= 0
    trk_ph = jnp.dot(trkg_ref[...], s_ref[...],
                     preferred_element_type=jnp.float32)
    det = detg_ref[...]
    det_ph = jnp.dot(det[:, :4], s4_ref[...],
                     preferred_element_type=jnp.float32)
    phases = jnp.where(sel, det_ph, trk_ph)
    out_ref[:, :EMBED] = jnp.cos(phases - shift_ref[...])
    meta_nat = metag_ref[...]
    meta_det = jnp.broadcast_to(det[:, 4:5], meta_nat.shape)
    out_ref[:, EMBED:] = jnp.where(sel, meta_det, meta_nat)


def _sine_encode(trkg, metag, detg, posq2d):
    n = trkg.shape[0]
    grid = n // ROWS_PER_BLOCK
    return pl.pallas_call(
        _encode_body,
        grid=(grid,),
        in_specs=[
            pl.BlockSpec((ROWS_PER_BLOCK, 4 * FRAME_RANGE), lambda i: (i, 0)),
            pl.BlockSpec((ROWS_PER_BLOCK, NMETA), lambda i: (i, 0)),
            pl.BlockSpec((ROWS_PER_BLOCK, 8), lambda i: (i, 0)),
            pl.BlockSpec((ROWS_PER_BLOCK, 1), lambda i: (i, 0)),
            pl.BlockSpec((4 * FRAME_RANGE, EMBED), lambda i: (0, 0)),
            pl.BlockSpec((4, EMBED), lambda i: (0, 0)),
            pl.BlockSpec((1, EMBED), lambda i: (0, 0)),
        ],
        out_specs=pl.BlockSpec((ROWS_PER_BLOCK, OUT_W), lambda i: (i, 0)),
        out_shape=jax.ShapeDtypeStruct((n, OUT_W), jnp.float32),
    )(trkg, metag, detg, posq2d, jnp.asarray(_S), jnp.asarray(_S4),
      jnp.asarray(_SHIFT))


def kernel(tracklets, tracklet_metadata, detections, write_indices, query_indices):
    trk1 = tracklets.reshape(_M * 4 * FRAME_RANGE)
    meta1 = tracklet_metadata.reshape(_M * NMETA)
    det8 = jnp.pad(detections, ((0, 0), (0, 3)))
    pmap = _posmap(write_indices)
    posq, trkg, metag, detg = _gatherq(
        pmap, query_indices, trk1, meta1, det8)
    return _sine_encode(trkg.reshape(B, 4 * FRAME_RANGE),
                        metag.reshape(B, NMETA), detg, posq.reshape(-1, 1))


# speedup vs baseline: 3.9674x; 3.9674x over previous
import functools

import jax
import jax.numpy as jnp
import numpy as np
from jax import lax
from jax.experimental import pallas as pl
from jax.experimental.pallas import tpu as pltpu
from jax.experimental.pallas import tpu_sc as plsc

FRAME_RANGE = 5
NUM_POS_FEATS = 32
DIM_METADATA = 1
TEMPERATURE = 10000.0
B = 16384
EMBED = FRAME_RANGE * 4 * NUM_POS_FEATS
NMETA = FRAME_RANGE * DIM_METADATA
OUT_W = EMBED + NMETA
ROWS_PER_BLOCK = 512

_NC, _NS, _L = 2, 16, 16
_NW = _NC * _NS
_M = 1000000
_RNG = 31264
_MPAD = _RNG * _NW
_NQ = B // _NW
_QCH = 128

_mesh = plsc.VectorSubcoreMesh(
    core_axis_name="c", subcore_axis_name="s", num_cores=_NC, num_subcores=_NS)
_scp = pltpu.CompilerParams(
    needs_layout_passes=False, use_tc_tiling_on_sc=False)


def _wid():
    return lax.axis_index("s") * _NC + lax.axis_index("c")


def _posmap_body(w_hbm, pmap_hbm, wv, buf, sem):
    wid = _wid()
    lo = wid * _RNG
    pltpu.async_copy(w_hbm, wv, sem).wait()
    lane = lax.iota(jnp.int32, _L)
    zeros = jnp.zeros((_L,), jnp.int32)
    onehot = jnp.int32(1) << lane

    def memset_it(r, c):
        buf[pl.ds(pl.multiple_of(r * _L, _L), _L)] = zeros
        return c

    lax.fori_loop(0, _RNG // _L, memset_it, 0)

    def scat_it(jc, c):
        wvv = wv[pl.ds(pl.multiple_of(jc * _L, _L), _L)]
        rel = wvv - lo
        valid = (rel >= 0) & (rel < _RNG)
        rel = jnp.where(valid, rel, 0)
        plsc.store_scatter(buf, [rel], (zeros + jc + 1) * 65536, mask=valid)
        plsc.addupdate_scatter(buf, [rel], onehot, mask=valid)
        return c

    lax.fori_loop(0, B // _L, scat_it, 0)
    pltpu.sync_copy(buf, pmap_hbm.at[pl.ds(lo, _RNG)])


@functools.partial(
    pl.kernel,
    out_type=jax.ShapeDtypeStruct((_MPAD,), jnp.int32),
    mesh=_mesh,
    compiler_params=_scp,
    scratch_types=[
        pltpu.VMEM((B,), jnp.int32),
        pltpu.VMEM((_RNG,), jnp.int32),
        pltpu.SemaphoreType.DMA,
    ],
)
def _posmap(w_hbm, pmap_hbm, wv, buf, sem):
    _posmap_body(w_hbm, pmap_hbm, wv, buf, sem)


def _gatherq_body(pmap_hbm, q_hbm, trk_hbm, meta_hbm, det_hbm,
                  posq_hbm, trkg_hbm, metag_hbm, detg_hbm,
                  qv, cmv, posv, pcv, trkv, metav, detv,
                  sem0, sem1, sem2, sem3):
    wid = _wid()
    base = wid * _NQ
    nch = _NQ // _QCH
    pltpu.async_copy(q_hbm.at[pl.ds(base, _NQ)], qv, sem0).wait()
    pos_cps = [
        pltpu.async_copy(
            pmap_hbm.at[qv.at[pl.ds(c * _QCH, _QCH)]],
            cmv.at[pl.ds(c * _QCH, _QCH)], sem0)
        for c in range(nch)
    ]
    trk_cps = [
        pltpu.async_copy(
            trk_hbm.at[qv.at[pl.ds(c * _QCH, _QCH)]],
            trkv.at[pl.ds(c * _QCH, _QCH)], sem1)
        for c in range(nch)
    ]
    meta_cps = [
        pltpu.async_copy(
            meta_hbm.at[qv.at[pl.ds(c * _QCH, _QCH)]],
            metav.at[pl.ds(c * _QCH, _QCH)], sem2)
        for c in range(nch)
    ]
    for cp in pos_cps:
        cp.wait()

    def decode_it(c, carry):
        sl = pl.ds(pl.multiple_of(c * _L, _L), _L)
        pk = cmv[sl]
        jc = (pk >> 16) - 1
        m = pk & 0xFFFF
        mf = m.astype(jnp.float32)
        hb = (lax.bitcast_convert_type(mf, jnp.int32) >> 23) - 127
        pos = jnp.where(jc >= 0, jc * _L + hb, -1)
        posv[sl] = pos
        pcv[sl] = jnp.maximum(pos, 0)
        return carry

    lax.fori_loop(0, _NQ // _L, decode_it, 0)
    det_cps = [
        pltpu.async_copy(
            det_hbm.at[pcv.at[pl.ds(c * _QCH, _QCH)]],
            detv.at[pl.ds(c * _QCH, _QCH)], sem3)
        for c in range(nch)
    ]
    pltpu.sync_copy(posv, posq_hbm.at[pl.ds(base, _NQ)])
    for cp in trk_cps:
        cp.wait()
    pltpu.sync_copy(trkv, trkg_hbm.at[pl.ds(base, _NQ)])
    for cp in meta_cps:
        cp.wait()
    pltpu.sync_copy(metav, metag_hbm.at[pl.ds(base, _NQ)])
    for cp in det_cps:
        cp.wait()
    pltpu.sync_copy(detv, detg_hbm.at[pl.ds(base, _NQ)])


@functools.partial(
    pl.kernel,
    out_type=(
        jax.ShapeDtypeStruct((B,), jnp.int32),
        jax.ShapeDtypeStruct((B, 24), jnp.float32),
        jax.ShapeDtypeStruct((B, 8), jnp.float32),
        jax.ShapeDtypeStruct((B, 8), jnp.float32),
    ),
    mesh=_mesh,
    compiler_params=_scp,
    scratch_types=[
        pltpu.VMEM((_NQ,), jnp.int32),
        pltpu.VMEM((_NQ,), jnp.int32),
        pltpu.VMEM((_NQ,), jnp.int32),
        pltpu.VMEM((_NQ,), jnp.int32),
        pltpu.VMEM((_NQ, 24), jnp.float32),
        pltpu.VMEM((_NQ, 8), jnp.float32),
        pltpu.VMEM((_NQ, 8), jnp.float32),
        pltpu.SemaphoreType.DMA,
        pltpu.SemaphoreType.DMA,
        pltpu.SemaphoreType.DMA,
        pltpu.SemaphoreType.DMA,
    ],
)
def _gatherq(pmap_hbm, q_hbm, trk_hbm, meta_hbm, det_hbm,
             posq_hbm, trkg_hbm, metag_hbm, detg_hbm,
             qv, cmv, posv, pcv, trkv, metav, detv,
             sem0, sem1, sem2, sem3):
    _gatherq_body(pmap_hbm, q_hbm, trk_hbm, meta_hbm, det_hbm,
                  posq_hbm, trkg_hbm, metag_hbm, detg_hbm,
                  qv, cmv, posv, pcv, trkv, metav, detv,
                  sem0, sem1, sem2, sem3)


def _freq_matrices():
    i = np.arange(NUM_POS_FEATS // 2, dtype=np.float64)
    w = 2.0 * np.pi * TEMPERATURE ** (-i / (NUM_POS_FEATS // 2))
    w2 = np.concatenate([w, w])
    ngroups = FRAME_RANGE * 4
    S = np.zeros((24, EMBED), dtype=np.float32)
    for g in range(ngroups):
        S[g, g * 32:(g + 1) * 32] = w2
    S4 = S[:20].reshape(FRAME_RANGE, 4, EMBED).sum(axis=0).astype(np.float32)
    shift = np.tile(
        np.concatenate([np.zeros(16), np.full(16, np.pi / 2.0)]), ngroups
    ).astype(np.float32)
    E5 = np.zeros((NMETA, 8), dtype=np.float32)
    for k in range(NMETA):
        E5[k, k] = 1.0
    E1 = np.zeros((1, 8), dtype=np.float32)
    E1[0, 4] = 1.0
    return S, S4, shift[:, None], E5, E1


_S, _S4, _SHIFT_T, _E5, _E1 = _freq_matrices()


def _encode_body(trkg_ref, metag_ref, detg_ref, posq_ref,
                 s_ref, s4_ref, shift_ref, e5_ref, e1_ref, out_ref):
    sel = posq_ref[...] >= 0
    dn = (((0,), (1,)), ((), ()))
    trk_ph = lax.dot_general(s_ref[...], trkg_ref[...], dn,
                             preferred_element_type=jnp.float32)
    det_ph = lax.dot_general(s4_ref[...], detg_ref[:, :4], dn,
                             preferred_element_type=jnp.float32)
    phases = jnp.where(sel, det_ph, trk_ph)
    out_ref[:EMBED, :] = jnp.cos(phases - shift_ref[...])
    dn2 = (((1,), (1,)), ((), ()))
    meta_nat = lax.dot_general(e5_ref[...], metag_ref[...], dn2,
                               preferred_element_type=jnp.float32)
    conf = lax.dot_general(e1_ref[...], detg_ref[...], dn2,
                           preferred_element_type=jnp.float32)
    out_ref[EMBED:, :] = jnp.where(
        sel, jnp.broadcast_to(conf, meta_nat.shape), meta_nat)


def _sine_encode(trkg, metag, detg, posq1r):
    n = trkg.shape[0]
    grid = n // ROWS_PER_BLOCK
    out_t = pl.pallas_call(
        _encode_body,
        grid=(grid,),
        in_specs=[
            pl.BlockSpec((ROWS_PER_BLOCK, 24), lambda i: (i, 0)),
            pl.BlockSpec((ROWS_PER_BLOCK, 8), lambda i: (i, 0)),
            pl.BlockSpec((ROWS_PER_BLOCK, 8), lambda i: (i, 0)),
            pl.BlockSpec((1, ROWS_PER_BLOCK), lambda i: (0, i)),
            pl.BlockSpec((24, EMBED), lambda i: (0, 0)),
            pl.BlockSpec((4, EMBED), lambda i: (0, 0)),
            pl.BlockSpec((EMBED, 1), lambda i: (0, 0)),
            pl.BlockSpec((NMETA, 8), lambda i: (0, 0)),
            pl.BlockSpec((1, 8), lambda i: (0, 0)),
        ],
        out_specs=pl.BlockSpec((OUT_W, ROWS_PER_BLOCK), lambda i: (0, i)),
        out_shape=jax.ShapeDtypeStruct((OUT_W, n), jnp.float32),
    )(trkg, metag, detg, posq1r, jnp.asarray(_S), jnp.asarray(_S4),
      jnp.asarray(_SHIFT_T), jnp.asarray(_E5), jnp.asarray(_E1))
    return out_t.T


def kernel(tracklets, tracklet_metadata, detections, write_indices, query_indices):
    trk24 = jnp.pad(tracklets.reshape(_M, 4 * FRAME_RANGE), ((0, 0), (0, 4)))
    meta8 = jnp.pad(tracklet_metadata.reshape(_M, NMETA), ((0, 0), (0, 3)))
    det8 = jnp.pad(detections, ((0, 0), (0, 3)))
    pmap = _posmap(write_indices)
    posq, trkg, metag, detg = _gatherq(
        pmap, query_indices, trk24, meta8, det8)
    return _sine_encode(trkg, metag, detg, posq.reshape(1, B))
